# baseline (speedup 1.0000x reference)
import functools

import jax
import jax.numpy as jnp
from jax import lax
from jax.experimental import pallas as pl
from jax.experimental.pallas import tpu as pltpu
from jax.experimental.pallas import tpu_sc as plsc

NC = 2
NS = 16
K = 128
LANES = 16


def _sc_mesh():
  return plsc.VectorSubcoreMesh(core_axis_name="c", subcore_axis_name="s")


def _deg_kernel(NPAD, C):
  PS = NPAD // NS

  @functools.partial(
      pl.kernel,
      out_type=jax.ShapeDtypeStruct((NS, PS), jnp.float32),
      mesh=_sc_mesh(),
      scratch_types=[
          pltpu.VMEM((C, K), jnp.int32),
          pltpu.VMEM((K,), jnp.float32),
          pltpu.VMEM((PS,), jnp.float32),
          pltpu.SemaphoreType.DMA,
          pltpu.VMEM_SHARED((NPAD,), jnp.float32),
      ],
  )
  def deg_kernel(dst_hbm, out_hbm, didx_v, ones_v, zbuf_v, sem, deg_sh):
    cid = lax.axis_index("c")
    sid = lax.axis_index("s")
    cnt = jnp.where(cid == 0, C, 0)

    pltpu.async_copy(dst_hbm.at[sid], didx_v, sem)

    for i in range(K // LANES):
      ones_v[pl.ds(i * LANES, LANES)] = jnp.full((LANES,), 1.0, jnp.float32)

    def _zinit(i, carry):
      zbuf_v[pl.ds(i * LANES, LANES)] = jnp.zeros((LANES,), jnp.float32)
      return carry

    lax.fori_loop(0, PS // LANES, _zinit, 0)
    pltpu.sync_copy(zbuf_v, deg_sh.at[pl.ds(sid * PS, PS)])
    pltpu.make_async_copy(dst_hbm.at[sid], didx_v, sem).wait()
    plsc.subcore_barrier()

    def _fire(j, carry):
      pltpu.async_copy(ones_v, deg_sh.at[didx_v.at[j]], sem, add=True)
      return carry

    lax.fori_loop(0, cnt, _fire, 0)

    def _drain(j, carry):
      pltpu.make_async_copy(ones_v, deg_sh.at[didx_v.at[j]], sem).wait()
      return carry

    lax.fori_loop(0, cnt, _drain, 0)
    plsc.subcore_barrier()

    @pl.when(cid == 0)
    def _():
      pltpu.sync_copy(deg_sh.at[pl.ds(sid * PS, PS)], out_hbm.at[sid])

  return deg_kernel


def _edge_pass_kernel(NPAD, C):
  @functools.partial(
      pl.kernel,
      out_type=jax.ShapeDtypeStruct((NPAD, 128), jnp.float32),
      mesh=_sc_mesh(),
      scratch_types=[
          [pltpu.VMEM((8, K), jnp.int32)] * 4,
          [pltpu.VMEM((8, K), jnp.int32)] * 4,
          pltpu.VMEM((K, 128), jnp.float32),
          pltpu.VMEM((K, 128), jnp.float32),
          [pltpu.SemaphoreType.DMA] * 4,
          [pltpu.SemaphoreType.DMA] * 4,
          pltpu.SemaphoreType.DMA,
          pltpu.SemaphoreType.DMA,
          pltpu.VMEM_SHARED((NPAD, 128), jnp.float32),
      ],
  )
  def edge_pass(g_hbm, src_hbm, dst_hbm, out_hbm, sidx, didx, rows0_v,
                rows1_v, ssems, dsems, gsem0, gsem1, acc_sh):
    cid = lax.axis_index("c")
    sid = lax.axis_index("s")
    PS = NPAD // NS
    rows = (rows0_v, rows1_v)
    gsems = (gsem0, gsem1)

    cnt = jnp.where(cid == 0, C, 4)
    base = jnp.where(cid == 0, sid * C, 0)

    def _start_sidx(j, q):
      pltpu.async_copy(src_hbm.at[base + j], sidx[q].at[pl.ds(0, 1)],
                       ssems[q])

    def _wait_sidx(j, q):
      pltpu.make_async_copy(src_hbm.at[base + j], sidx[q].at[pl.ds(0, 1)],
                            ssems[q]).wait()

    def _start_didx(j, q):
      pltpu.async_copy(dst_hbm.at[base + j], didx[q].at[pl.ds(0, 1)],
                       dsems[q])

    def _wait_didx(j, q):
      pltpu.make_async_copy(dst_hbm.at[base + j], didx[q].at[pl.ds(0, 1)],
                            dsems[q]).wait()

    def _start_gather(j, q, p):
      pltpu.async_copy(g_hbm.at[sidx[q].at[0]], rows[p], gsems[p])

    def _wait_gather(j, q, p):
      pltpu.make_async_copy(g_hbm.at[sidx[q].at[0]], rows[p],
                            gsems[p]).wait()

    def _scatter(j, q, p):
      pltpu.sync_copy(rows[p], acc_sh.at[didx[q].at[0]], add=True)

    for q in range(4):
      _start_sidx(q, q)
      _start_didx(q, q)

    def _zero(r, carry):
      for cidx in range(128 // LANES):
        rows0_v[r, pl.ds(cidx * LANES, LANES)] = jnp.zeros((LANES,),
                                                           jnp.float32)
      return carry

    lax.fori_loop(0, K, _zero, 0)
    for t in range(PS // K):
      pltpu.sync_copy(rows0_v, acc_sh.at[pl.ds(sid * PS + t * K, K), :])
    plsc.subcore_barrier()

    _wait_sidx(0, 0)
    _start_gather(0, 0, 0)
    _wait_sidx(1, 1)
    _start_gather(1, 1, 1)

    def _quad(t, carry):
      j0 = 4 * t
      for u, (q, p) in enumerate(((0, 0), (1, 1), (2, 0), (3, 1))):
        j = j0 + u
        _wait_gather(j, q, p)
        _wait_didx(j, q)
        _scatter(j, q, p)
        _start_sidx(j + 4, q)
        _start_didx(j + 4, q)
        _wait_sidx(j + 2, (q + 2) % 4)
        _start_gather(j + 2, (q + 2) % 4, p)
      return carry

    lax.fori_loop(0, (cnt - 4) // 4, _quad, 0)
    _wait_gather(cnt - 4, 0, 0)
    _wait_didx(cnt - 4, 0)
    _scatter(cnt - 4, 0, 0)
    _wait_sidx(cnt - 2, 2)
    _start_gather(cnt - 2, 2, 0)
    _wait_gather(cnt - 3, 1, 1)
    _wait_didx(cnt - 3, 1)
    _scatter(cnt - 3, 1, 1)
    _wait_sidx(cnt - 1, 3)
    _start_gather(cnt - 1, 3, 1)
    _wait_gather(cnt - 2, 2, 0)
    _wait_didx(cnt - 2, 2)
    _scatter(cnt - 2, 2, 0)
    _wait_gather(cnt - 1, 3, 1)
    _wait_didx(cnt - 1, 3)
    _scatter(cnt - 1, 3, 1)

    plsc.subcore_barrier()

    @pl.when(cid == 0)
    def _():
      for t in range(PS // K):
        pltpu.async_copy(acc_sh.at[pl.ds(sid * PS + t * K, K), :],
                         out_hbm.at[pl.ds(sid * PS + t * K, K), :], gsem0)
      for t in range(PS // K):
        pltpu.make_async_copy(acc_sh.at[pl.ds(sid * PS + t * K, K), :],
                              out_hbm.at[pl.ds(sid * PS + t * K, K), :],
                              gsem0).wait()

  return edge_pass


def _dinv(deg_ref):
  return lax.rsqrt(deg_ref[...] + 1.0)


def _gelu(s):
  return 0.5 * s * (1.0 + lax.erf(s * 0.7071067811865476))


def _tc_first(deg2, x_pad, W1, NPAD, R):
  def body(deg_ref, x_ref, w_ref, g_ref):
    h = jnp.dot(x_ref[...], w_ref[...], preferred_element_type=jnp.float32)
    g_ref[...] = h * _dinv(deg_ref)

  return pl.pallas_call(
      body,
      grid=(NPAD // R,),
      in_specs=[
          pl.BlockSpec((R, 1), lambda i: (i, 0)),
          pl.BlockSpec((R, 128), lambda i: (i, 0)),
          pl.BlockSpec((128, 128), lambda i: (0, 0)),
      ],
      out_specs=pl.BlockSpec((R, 128), lambda i: (i, 0)),
      out_shape=jax.ShapeDtypeStruct((NPAD, 128), jnp.float32),
  )(deg2, x_pad, W1)


def _tc_mid(acc, g1, deg2, b1, W2, NPAD, R):
  def body(acc_ref, g_ref, deg_ref, b_ref, w_ref, out_ref):
    dinv = _dinv(deg_ref)
    s = (acc_ref[...] + g_ref[...]) * dinv + b_ref[...]
    h2 = jnp.dot(_gelu(s), w_ref[...], preferred_element_type=jnp.float32)
    out_ref[...] = h2 * dinv

  return pl.pallas_call(
      body,
      grid=(NPAD // R,),
      in_specs=[
          pl.BlockSpec((R, 128), lambda i: (i, 0)),
          pl.BlockSpec((R, 128), lambda i: (i, 0)),
          pl.BlockSpec((R, 1), lambda i: (i, 0)),
          pl.BlockSpec((1, 128), lambda i: (0, 0)),
          pl.BlockSpec((128, 128), lambda i: (0, 0)),
      ],
      out_specs=pl.BlockSpec((R, 128), lambda i: (i, 0)),
      out_shape=jax.ShapeDtypeStruct((NPAD, 128), jnp.float32),
  )(acc, g1, deg2, b1, W2)


def _tc_last(acc, g2, deg2, b2, NPAD, R):
  def body(acc_ref, g_ref, deg_ref, b_ref, out_ref):
    s = (acc_ref[...] + g_ref[...]) * _dinv(deg_ref) + b_ref[...]
    out_ref[...] = _gelu(s)

  return pl.pallas_call(
      body,
      grid=(NPAD // R,),
      in_specs=[
          pl.BlockSpec((R, 128), lambda i: (i, 0)),
          pl.BlockSpec((R, 128), lambda i: (i, 0)),
          pl.BlockSpec((R, 1), lambda i: (i, 0)),
          pl.BlockSpec((1, 128), lambda i: (0, 0)),
      ],
      out_specs=pl.BlockSpec((R, 128), lambda i: (i, 0)),
      out_shape=jax.ShapeDtypeStruct((NPAD, 128), jnp.float32),
  )(acc, g2, deg2, b2)


def kernel(x, edge_index, W1, b1, W2, b2):
  B, T, J, D = x.shape
  N = B * T * J
  E = edge_index.shape[1]
  R = 1280
  NPAD = -(-(N + 1) // R) * R
  C = -(-(-(-E // (NS * K))) // 4) * 4
  E_pad = NS * C * K
  TOT = NS * C

  src = jnp.pad(edge_index[0], (0, E_pad - E)).reshape(TOT, 1, K)
  dst_flat = jnp.pad(edge_index[1], (0, E_pad - E),
                     constant_values=N)
  dst = dst_flat.reshape(TOT, 1, K)
  dst_deg = dst_flat.reshape(NS, C, K)
  x_flat = jnp.pad(x.reshape(N, D), ((0, NPAD - N), (0, 0)))

  deg = _deg_kernel(NPAD, C)(dst_deg).reshape(NPAD)
  deg2 = deg.reshape(NPAD, 1)
  edge_pass = _edge_pass_kernel(NPAD, C)

  g1 = _tc_first(deg2, x_flat, W1, NPAD, R)
  acc1 = edge_pass(g1, src, dst)
  g2 = _tc_mid(acc1, g1, deg2, b1.reshape(1, 128), W2, NPAD, R)
  acc2 = edge_pass(g2, src, dst)
  out = _tc_last(acc2, g2, deg2, b2.reshape(1, 128), NPAD, R)
  return out[:N].reshape(B, T, J, 128)


# speedup vs baseline: 1.8098x; 1.8098x over previous
import functools

import jax
import jax.numpy as jnp
from jax import lax
from jax.experimental import pallas as pl
from jax.experimental.pallas import tpu as pltpu
from jax.experimental.pallas import tpu_sc as plsc

NC = 2
NS = 16
NW = NC * NS
K = 128
LANES = 16


def _sc_mesh():
  return plsc.VectorSubcoreMesh(core_axis_name="c", subcore_axis_name="s")


def _deg_kernel(NPAD, C):
  @functools.partial(
      pl.kernel,
      out_type=jax.ShapeDtypeStruct((NC, NPAD), jnp.float32),
      mesh=_sc_mesh(),
      scratch_types=[
          pltpu.VMEM((C, K), jnp.int32),
          pltpu.VMEM((K,), jnp.float32),
          pltpu.VMEM((NPAD // NS,), jnp.float32),
          pltpu.SemaphoreType.DMA,
          pltpu.VMEM_SHARED((NPAD,), jnp.float32),
      ],
  )
  def deg_kernel(dst_hbm, out_hbm, didx_v, ones_v, zbuf_v, sem, deg_sh):
    cid = lax.axis_index("c")
    sid = lax.axis_index("s")
    wid = cid * NS + sid
    PS = NPAD // NS

    pltpu.async_copy(dst_hbm.at[wid], didx_v, sem)

    for i in range(K // LANES):
      ones_v[pl.ds(i * LANES, LANES)] = jnp.full((LANES,), 1.0, jnp.float32)

    def _zinit(i, carry):
      zbuf_v[pl.ds(i * LANES, LANES)] = jnp.zeros((LANES,), jnp.float32)
      return carry

    lax.fori_loop(0, PS // LANES, _zinit, 0)
    pltpu.sync_copy(zbuf_v, deg_sh.at[pl.ds(sid * PS, PS)])
    pltpu.make_async_copy(dst_hbm.at[wid], didx_v, sem).wait()
    plsc.subcore_barrier()

    def _fire(j, carry):
      pltpu.async_copy(ones_v, deg_sh.at[didx_v.at[j]], sem, add=True)
      return carry

    lax.fori_loop(0, C, _fire, 0)

    def _drain(j, carry):
      pltpu.make_async_copy(ones_v, deg_sh.at[didx_v.at[j]], sem).wait()
      return carry

    lax.fori_loop(0, C, _drain, 0)
    plsc.subcore_barrier()
    pltpu.sync_copy(deg_sh.at[pl.ds(sid * PS, PS)],
                    out_hbm.at[cid, pl.ds(sid * PS, PS)])

  return deg_kernel


def _edge_pass_kernel(NPAD, C):
  @functools.partial(
      pl.kernel,
      out_type=jax.ShapeDtypeStruct((NC, NPAD, 128), jnp.float32),
      mesh=_sc_mesh(),
      scratch_types=[
          pltpu.VMEM((C, K), jnp.int32),
          pltpu.VMEM((1, K), jnp.int32),
          pltpu.VMEM((1, K), jnp.int32),
          pltpu.VMEM((K, 128), jnp.float32),
          pltpu.VMEM((K, 128), jnp.float32),
          pltpu.SemaphoreType.DMA,
          pltpu.SemaphoreType.DMA,
          pltpu.SemaphoreType.DMA,
          pltpu.SemaphoreType.DMA,
          pltpu.VMEM_SHARED((NPAD, 128), jnp.float32),
      ],
  )
  def edge_pass(g_hbm, src_hbm, dst_hbm, out_hbm, sidx_v, didx0_v, didx1_v,
                rows0_v, rows1_v, gsem0, gsem1, dsem0, dsem1, acc_sh):
    cid = lax.axis_index("c")
    sid = lax.axis_index("s")
    wid = cid * NS + sid
    PS = NPAD // NS
    rows = (rows0_v, rows1_v)
    gsems = (gsem0, gsem1)
    didx = (didx0_v, didx1_v)
    dsems = (dsem0, dsem1)

    pltpu.async_copy(src_hbm.at[wid], sidx_v, gsem0)

    def _zero(r, carry):
      for cidx in range(128 // LANES):
        rows0_v[r, pl.ds(cidx * LANES, LANES)] = jnp.zeros((LANES,),
                                                           jnp.float32)
      return carry

    lax.fori_loop(0, K, _zero, 0)
    for t in range(PS // K):
      pltpu.sync_copy(rows0_v, acc_sh.at[pl.ds(sid * PS + t * K, K), :])
    pltpu.make_async_copy(src_hbm.at[wid], sidx_v, gsem0).wait()
    plsc.subcore_barrier()

    def _start_gather(j, p):
      pltpu.async_copy(g_hbm.at[sidx_v.at[j]], rows[p], gsems[p])

    def _wait_gather(j, p):
      pltpu.make_async_copy(g_hbm.at[sidx_v.at[j]], rows[p], gsems[p]).wait()

    def _start_didx(j, p):
      pltpu.async_copy(dst_hbm.at[wid * C + j], didx[p], dsems[p])

    def _wait_didx(j, p):
      pltpu.make_async_copy(dst_hbm.at[wid * C + j], didx[p], dsems[p]).wait()

    def _scatter(j, p):
      pltpu.sync_copy(rows[p], acc_sh.at[didx[p].at[0]], add=True)

    _start_didx(0, 0)
    _start_gather(0, 0)
    _start_didx(1, 1)

    def _pair(t, carry):
      j0 = 2 * t
      _start_gather(j0 + 1, 1)
      _wait_gather(j0, 0)
      _wait_didx(j0, 0)
      _scatter(j0, 0)
      _start_gather(j0 + 2, 0)
      _start_didx(j0 + 2, 0)
      _wait_gather(j0 + 1, 1)
      _wait_didx(j0 + 1, 1)
      _scatter(j0 + 1, 1)

      @pl.when(j0 + 3 < C)
      def _():
        _start_didx(j0 + 3, 1)

      return carry

    if C % 2 == 1:
      lax.fori_loop(0, (C - 1) // 2, _pair, 0)
      _wait_gather(C - 1, 0)
      _wait_didx(C - 1, 0)
      _scatter(C - 1, 0)
    else:
      lax.fori_loop(0, (C - 2) // 2, _pair, 0)
      _start_gather(C - 1, 1)
      _wait_gather(C - 2, 0)
      _wait_didx(C - 2, 0)
      _scatter(C - 2, 0)
      _wait_gather(C - 1, 1)
      _wait_didx(C - 1, 1)
      _scatter(C - 1, 1)

    plsc.subcore_barrier()
    for t in range(PS // K):
      pltpu.async_copy(acc_sh.at[pl.ds(sid * PS + t * K, K), :],
                       out_hbm.at[cid, pl.ds(sid * PS + t * K, K), :], gsem0)
    for t in range(PS // K):
      pltpu.make_async_copy(acc_sh.at[pl.ds(sid * PS + t * K, K), :],
                            out_hbm.at[cid, pl.ds(sid * PS + t * K, K), :],
                            gsem0).wait()

  return edge_pass


def _dinv(deg_ref):
  return lax.rsqrt(deg_ref[0] + deg_ref[1] + 1.0)


def _gelu(s):
  return 0.5 * s * (1.0 + lax.erf(s * 0.7071067811865476))


def _tc_first(deg3, x_pad, W1, NPAD, R):
  def body(deg_ref, x_ref, w_ref, g_ref):
    h = jnp.dot(x_ref[...], w_ref[...], preferred_element_type=jnp.float32)
    g_ref[...] = h * _dinv(deg_ref)

  return pl.pallas_call(
      body,
      grid=(NPAD // R,),
      in_specs=[
          pl.BlockSpec((2, R, 1), lambda i: (0, i, 0)),
          pl.BlockSpec((R, 128), lambda i: (i, 0)),
          pl.BlockSpec((128, 128), lambda i: (0, 0)),
      ],
      out_specs=pl.BlockSpec((R, 128), lambda i: (i, 0)),
      out_shape=jax.ShapeDtypeStruct((NPAD, 128), jnp.float32),
  )(deg3, x_pad, W1)


def _tc_mid(acc, g1, deg3, b1, W2, NPAD, R):
  def body(acc_ref, g_ref, deg_ref, b_ref, w_ref, out_ref):
    dinv = _dinv(deg_ref)
    s = (acc_ref[0] + acc_ref[1] + g_ref[...]) * dinv + b_ref[...]
    h2 = jnp.dot(_gelu(s), w_ref[...], preferred_element_type=jnp.float32)
    out_ref[...] = h2 * dinv

  return pl.pallas_call(
      body,
      grid=(NPAD // R,),
      in_specs=[
          pl.BlockSpec((2, R, 128), lambda i: (0, i, 0)),
          pl.BlockSpec((R, 128), lambda i: (i, 0)),
          pl.BlockSpec((2, R, 1), lambda i: (0, i, 0)),
          pl.BlockSpec((1, 128), lambda i: (0, 0)),
          pl.BlockSpec((128, 128), lambda i: (0, 0)),
      ],
      out_specs=pl.BlockSpec((R, 128), lambda i: (i, 0)),
      out_shape=jax.ShapeDtypeStruct((NPAD, 128), jnp.float32),
  )(acc, g1, deg3, b1, W2)


def _tc_last(acc, g2, deg3, b2, NPAD, R):
  def body(acc_ref, g_ref, deg_ref, b_ref, out_ref):
    s = (acc_ref[0] + acc_ref[1] + g_ref[...]) * _dinv(deg_ref) + b_ref[...]
    out_ref[...] = _gelu(s)

  return pl.pallas_call(
      body,
      grid=(NPAD // R,),
      in_specs=[
          pl.BlockSpec((2, R, 128), lambda i: (0, i, 0)),
          pl.BlockSpec((R, 128), lambda i: (i, 0)),
          pl.BlockSpec((2, R, 1), lambda i: (0, i, 0)),
          pl.BlockSpec((1, 128), lambda i: (0, 0)),
      ],
      out_specs=pl.BlockSpec((R, 128), lambda i: (i, 0)),
      out_shape=jax.ShapeDtypeStruct((NPAD, 128), jnp.float32),
  )(acc, g2, deg3, b2)


def kernel(x, edge_index, W1, b1, W2, b2):
  B, T, J, D = x.shape
  N = B * T * J
  E = edge_index.shape[1]
  R = 1280
  NPAD = -(-(N + 1) // R) * R
  C = -(-E // (NW * K))
  E_pad = NW * K * C

  src = jnp.pad(edge_index[0], (0, E_pad - E)).reshape(NW, C, K)
  dst_flat = jnp.pad(edge_index[1], (0, E_pad - E),
                     constant_values=N)
  dst_deg = dst_flat.reshape(NW, C, K)
  dst = dst_flat.reshape(NW * C, 1, K)
  x_flat = jnp.pad(x.reshape(N, D), ((0, NPAD - N), (0, 0)))

  deg = _deg_kernel(NPAD, C)(dst_deg)
  deg3 = deg.reshape(NC, NPAD, 1)
  edge_pass = _edge_pass_kernel(NPAD, C)

  g1 = _tc_first(deg3, x_flat, W1, NPAD, R)
  acc1 = edge_pass(g1, src, dst)
  g2 = _tc_mid(acc1, g1, deg3, b1.reshape(1, 128), W2, NPAD, R)
  acc2 = edge_pass(g2, src, dst)
  out = _tc_last(acc2, g2, deg3, b2.reshape(1, 128), NPAD, R)
  return out[:N].reshape(B, T, J, 128)
